# Initial kernel scaffold; baseline (speedup 1.0000x reference)
import jax
import jax.numpy as jnp
from jax.experimental import pallas as pl


def kernel(src, table, W, b):
    raise NotImplementedError("write your pallas kernel here")


# speedup vs baseline: 1.1896x; 1.1896x over previous
import functools

import jax
import jax.numpy as jnp
from jax import lax
from jax.experimental import pallas as pl
from jax.experimental.pallas import tpu as pltpu
from jax.experimental.pallas import tpu_sc as plsc

B = 16384
L = 50
D = 64
DO = 64
N = B * L

NC = 2
NS = 16
NW = NC * NS
PER_W = N // NW
CHUNK = 512
NCHUNK = PER_W // CHUNK


def _gather_body(idx_hbm, table_hbm, out_hbm, idx_v, rows_v, sems):
    wid = lax.axis_index("s") * NC + lax.axis_index("c")
    base = wid * PER_W

    pltpu.sync_copy(idx_hbm.at[pl.ds(base, CHUNK)], idx_v.at[0])
    pltpu.async_copy(table_hbm.at[idx_v.at[0]], rows_v.at[0], sems.at[0])

    def step(i, _):
        cur = lax.rem(i, 2)
        nxt = lax.rem(i + 1, 2)

        @pl.when(i + 1 < NCHUNK)
        def _():
            off = base + (i + 1) * CHUNK
            pltpu.sync_copy(idx_hbm.at[pl.ds(off, CHUNK)], idx_v.at[nxt])
            pltpu.async_copy(table_hbm.at[idx_v.at[nxt]], rows_v.at[nxt],
                             sems.at[nxt])

        pltpu.make_async_copy(table_hbm.at[idx_v.at[cur]], rows_v.at[cur],
                              sems.at[cur]).wait()
        pltpu.sync_copy(rows_v.at[cur], out_hbm.at[pl.ds(base + i * CHUNK, CHUNK)])
        return ()

    lax.fori_loop(0, NCHUNK, step, (), unroll=2)


def _sc_gather(idx, table):
    mesh = plsc.VectorSubcoreMesh(core_axis_name="c", subcore_axis_name="s")
    k = functools.partial(
        pl.kernel, mesh=mesh,
        out_type=jax.ShapeDtypeStruct((N, D), jnp.float32),
        scratch_types=[
            pltpu.VMEM((2, CHUNK), jnp.int32),
            pltpu.VMEM((2, CHUNK, D), jnp.float32),
            pltpu.SemaphoreType.DMA((2,)),
        ],
        compiler_params=pltpu.CompilerParams(use_tc_tiling_on_sc=False),
    )(_gather_body)
    return k(idx, table)


def _mm_body(x_ref, w_ref, b_ref, o_ref):
    o_ref[...] = jnp.dot(x_ref[...], w_ref[...],
                         preferred_element_type=jnp.float32) + b_ref[...]


def _tc_project(rows, Wt, b2):
    BR = 8192
    grid = N // BR
    return pl.pallas_call(
        _mm_body,
        grid=(grid,),
        in_specs=[
            pl.BlockSpec((BR, D), lambda i: (i, 0)),
            pl.BlockSpec((D, DO), lambda i: (0, 0)),
            pl.BlockSpec((1, DO), lambda i: (0, 0)),
        ],
        out_specs=pl.BlockSpec((BR, DO), lambda i: (i, 0)),
        out_shape=jax.ShapeDtypeStruct((N, DO), jnp.float32),
        compiler_params=pltpu.CompilerParams(
            dimension_semantics=("arbitrary",),
        ),
    )(rows, Wt, b2)


def kernel(src, table, W, b):
    idx = src.reshape(N).astype(jnp.int32)
    rows = _sc_gather(idx, table)
    out = _tc_project(rows, W.T, b.reshape(1, DO))
    return out.reshape(B, L, DO)


# speedup vs baseline: 1.5979x; 1.3433x over previous
import functools

import jax
import jax.numpy as jnp
from jax import lax
from jax.experimental import pallas as pl
from jax.experimental.pallas import tpu as pltpu
from jax.experimental.pallas import tpu_sc as plsc

B = 16384
L = 50
D = 64
DO = 64
N = B * L
LP = 56
N2 = B * LP

NC = 2
NS = 16
NW = NC * NS
B_PER_W = B // NW
GB = 16
CHUNK = GB * L
NCH = B_PER_W // GB


def _gather_body(idx_hbm, table_hbm, out_hbm, idx_v, rows_v, gsem, wsem):
    wid = lax.axis_index("s") * NC + lax.axis_index("c")
    b_base = wid * B_PER_W

    def idx_load(c, buf):
        pltpu.sync_copy(idx_hbm.at[pl.ds((b_base + c * GB) * L, CHUNK)],
                        idx_v.at[buf])

    def gather_start(c, buf):
        pltpu.async_copy(table_hbm.at[idx_v.at[buf]], rows_v.at[buf],
                         gsem.at[buf])

    def gather_wait(buf):
        pltpu.make_async_copy(table_hbm.at[idx_v.at[buf]], rows_v.at[buf],
                              gsem.at[buf]).wait()

    def wb_copy(c, buf, k):
        b0 = b_base + c * GB
        src = rows_v.at[buf].at[pl.ds(k * L, L), :]
        dst = out_hbm.at[pl.ds((b0 + k) * LP, L), pl.ds(0, D)]
        return pltpu.make_async_copy(src, dst, wsem.at[buf])

    idx_load(0, 0)
    gather_start(0, 0)

    def step(c, _):
        cur = lax.rem(c, 2)
        nxt = lax.rem(c + 1, 2)

        gather_wait(cur)

        @pl.when(c + 1 < NCH)
        def _():
            idx_load(c + 1, nxt)

        @pl.when(c >= 1)
        def _():
            for k in range(GB):
                wb_copy(c - 1, nxt, k).wait()

        @pl.when(c + 1 < NCH)
        def _():
            gather_start(c + 1, nxt)

        for k in range(GB):
            wb_copy(c, cur, k).start()
        return ()

    lax.fori_loop(0, NCH, step, ())
    last = NCH - 1
    for k in range(GB):
        wb_copy(last, lax.rem(last, 2), k).wait()


def _sc_gather(idx, table):
    mesh = plsc.VectorSubcoreMesh(core_axis_name="c", subcore_axis_name="s")
    k = functools.partial(
        pl.kernel, mesh=mesh,
        out_type=jax.ShapeDtypeStruct((N2, 128), jnp.float32),
        scratch_types=[
            pltpu.VMEM((2, CHUNK), jnp.int32),
            pltpu.VMEM((2, CHUNK, D), jnp.float32),
            pltpu.SemaphoreType.DMA((2,)),
            pltpu.SemaphoreType.DMA((2,)),
        ],
        compiler_params=pltpu.CompilerParams(use_tc_tiling_on_sc=False),
    )(_gather_body)
    return k(idx, table)


BB = 64
BR = BB * LP


def _mm_body(x_ref, w_ref, b_ref, o_ref):
    x = x_ref[...][:, :D]
    y = jnp.dot(x, w_ref[...], preferred_element_type=jnp.float32)
    y = y + b_ref[...]
    y3 = y.reshape(BB, LP, DO)
    o_ref[...] = y3[:, :L, :]


def _tc_project(x2, Wt, b2):
    grid = B // BB
    return pl.pallas_call(
        _mm_body,
        grid=(grid,),
        in_specs=[
            pl.BlockSpec((BR, 128), lambda i: (i, 0)),
            pl.BlockSpec((D, DO), lambda i: (0, 0)),
            pl.BlockSpec((1, DO), lambda i: (0, 0)),
        ],
        out_specs=pl.BlockSpec((BB, L, DO), lambda i: (i, 0, 0)),
        out_shape=jax.ShapeDtypeStruct((B, L, DO), jnp.float32),
        compiler_params=pltpu.CompilerParams(
            dimension_semantics=("arbitrary",),
        ),
    )(x2, Wt, b2)


def kernel(src, table, W, b):
    idx = src.reshape(N).astype(jnp.int32)
    x2 = _sc_gather(idx, table)
    return _tc_project(x2, W.T, b.reshape(1, DO))


# speedup vs baseline: 1.7858x; 1.1176x over previous
import functools

import jax
import jax.numpy as jnp
from jax import lax
from jax.experimental import pallas as pl
from jax.experimental.pallas import tpu as pltpu
from jax.experimental.pallas import tpu_sc as plsc

B = 16384
L = 50
D = 64
DO = 64
N = B * L
V = 1000000

NC = 2
NS = 16
NW = NC * NS
PER_W = N // NW
CHUNK = 512
NCH = PER_W // CHUNK

VH = V // 2
HB = 4000


def _proj_body(xlo_ref, xhi_ref, w_ref, b_ref, o_ref):
    ylo = jnp.dot(xlo_ref[...], w_ref[...], preferred_element_type=jnp.float32)
    yhi = jnp.dot(xhi_ref[...], w_ref[...], preferred_element_type=jnp.float32)
    bb = b_ref[...]
    o_ref[...] = jnp.concatenate([ylo + bb, yhi + bb], axis=1)


def _tc_project(table, Wt, b2):
    grid = VH // HB
    nblk = VH // HB
    return pl.pallas_call(
        _proj_body,
        grid=(grid,),
        in_specs=[
            pl.BlockSpec((HB, D), lambda i: (i, 0)),
            pl.BlockSpec((HB, D), lambda i: (i + nblk, 0)),
            pl.BlockSpec((D, DO), lambda i: (0, 0)),
            pl.BlockSpec((1, DO), lambda i: (0, 0)),
        ],
        out_specs=pl.BlockSpec((HB, 128), lambda i: (i, 0)),
        out_shape=jax.ShapeDtypeStruct((VH, 128), jnp.float32),
        compiler_params=pltpu.CompilerParams(
            dimension_semantics=("arbitrary",),
        ),
    )(table, table, Wt, b2)


def _gather_body(idx_hbm, table_hbm, out_hbm, idx_v, rows_v, gsem, wsem):
    wid = lax.axis_index("s") * NC + lax.axis_index("c")
    base = wid * PER_W

    def idx_load(c, buf):
        pltpu.sync_copy(idx_hbm.at[pl.ds(base + c * CHUNK, CHUNK)],
                        idx_v.at[buf])

    def gather(c, buf):
        return pltpu.make_async_copy(table_hbm.at[idx_v.at[buf]],
                                     rows_v.at[buf], gsem.at[buf])

    def wb(c, buf):
        return pltpu.make_async_copy(
            rows_v.at[buf],
            out_hbm.at[pl.ds(base + c * CHUNK, CHUNK), :],
            wsem.at[buf])

    idx_load(0, 0)
    gather(0, 0).start()

    def step(c, _):
        cur = lax.rem(c, 2)
        nxt = lax.rem(c + 1, 2)

        gather(c, cur).wait()

        @pl.when(c + 1 < NCH)
        def _():
            idx_load(c + 1, nxt)

        @pl.when(c >= 1)
        def _():
            wb(c - 1, nxt).wait()

        @pl.when(c + 1 < NCH)
        def _():
            gather(c + 1, nxt).start()

        wb(c, cur).start()
        return ()

    lax.fori_loop(0, NCH, step, ())
    wb(NCH - 1, lax.rem(NCH - 1, 2)).wait()


def _sc_gather(idx, tableL):
    mesh = plsc.VectorSubcoreMesh(core_axis_name="c", subcore_axis_name="s")
    k = functools.partial(
        pl.kernel, mesh=mesh,
        out_type=jax.ShapeDtypeStruct((N, D), jnp.float32),
        scratch_types=[
            pltpu.VMEM((2, CHUNK), jnp.int32),
            pltpu.VMEM((2, CHUNK, D), jnp.float32),
            pltpu.SemaphoreType.DMA((2,)),
            pltpu.SemaphoreType.DMA((2,)),
        ],
        compiler_params=pltpu.CompilerParams(use_tc_tiling_on_sc=False),
    )(_gather_body)
    return k(idx, tableL)


def kernel(src, table, W, b):
    idx = src.reshape(N).astype(jnp.int32)
    idx2 = jnp.where(idx < VH, 2 * idx, 2 * (idx - VH) + 1)
    tP = _tc_project(table, W.T, b.reshape(1, DO))
    tL = tP.reshape(V, D)
    out = _sc_gather(idx2, tL)
    return out.reshape(B, L, DO)
